# baseline (speedup 1.0000x reference)
import jax
import jax.numpy as jnp
from jax.experimental import pallas as pl
from jax.experimental.pallas import tpu as pltpu


def _body(pos_ref, hid_ref, mask_ref, emb_ref, out_ref, mask_out_ref, sem):
    B = hid_ref.shape[0]
    NCT = emb_ref.shape[1]
    p = pos_ref[0]

    bulk = pltpu.make_async_copy(hid_ref, out_ref.at[:, NCT:], sem.at[0])
    bulk.start()
    ctx_copies = []
    for b in range(B):
        cp = pltpu.make_async_copy(
            emb_ref.at[p], out_ref.at[b, :NCT], sem.at[1 + b]
        )
        cp.start()
        ctx_copies.append(cp)

    mask_out_ref[:, :NCT] = jnp.ones((B, NCT), mask_out_ref.dtype)
    mask_out_ref[:, NCT:] = mask_ref[:, :]

    bulk.wait()
    for cp in ctx_copies:
        cp.wait()


def kernel(hidden_states, attention_mask, embeddings, position):
    B, S, D = hidden_states.shape
    _, NCT, _ = embeddings.shape
    pos = jnp.asarray(position, jnp.int32).reshape((1,))
    hid4 = hidden_states.reshape(B, S, 8, D // 8)
    emb4 = embeddings.reshape(embeddings.shape[0], NCT, 8, D // 8)

    out_hid, out_mask = pl.pallas_call(
        _body,
        in_specs=[
            pl.BlockSpec(memory_space=pltpu.SMEM),
            pl.BlockSpec(memory_space=pl.ANY),
            pl.BlockSpec(memory_space=pltpu.VMEM),
            pl.BlockSpec(memory_space=pl.ANY),
        ],
        out_specs=[
            pl.BlockSpec(memory_space=pl.ANY),
            pl.BlockSpec(memory_space=pltpu.VMEM),
        ],
        out_shape=[
            jax.ShapeDtypeStruct((B, NCT + S, 8, D // 8), hidden_states.dtype),
            jax.ShapeDtypeStruct((B, NCT + S), attention_mask.dtype),
        ],
        scratch_shapes=[pltpu.SemaphoreType.DMA((1 + B,))],
    )(pos, hid4, attention_mask, emb4)
    return (out_hid.reshape(B, NCT + S, D), out_mask)
